# baseline (speedup 1.0000x reference)
import functools

import jax
import jax.numpy as jnp
from jax import lax
from jax.experimental import pallas as pl
from jax.experimental.pallas import tpu as pltpu
from jax.experimental.pallas import tpu_sc as plsc

SEQ_LEN = 200
EMBED_DIM = 32
NUM_CORES = 2
NUM_SUBCORES = 16
NW = NUM_CORES * NUM_SUBCORES
ROWS = 40


def _sc_embed(x_flat, tok_table, pos_table):
    n = x_flat.shape[0]
    per_w = n // NW
    nchunks = per_w // ROWS
    mesh = plsc.VectorSubcoreMesh(core_axis_name="c", subcore_axis_name="s")

    @functools.partial(
        pl.kernel,
        out_type=jax.ShapeDtypeStruct((n, EMBED_DIM), jnp.float32),
        mesh=mesh,
        compiler_params=pltpu.CompilerParams(use_tc_tiling_on_sc=False),
        scratch_types=[
            pltpu.VMEM((ROWS,), jnp.int32),
            pltpu.VMEM((ROWS, EMBED_DIM), jnp.float32),
            pltpu.VMEM((SEQ_LEN, EMBED_DIM), jnp.float32),
            pltpu.SemaphoreType.DMA,
        ],
    )
    def k(x_hbm, tok_hbm, pos_hbm, out_hbm, idx_v, rows_v, pos_v, sem):
        wid = lax.axis_index("s") * NUM_CORES + lax.axis_index("c")
        base = wid * per_w
        pltpu.sync_copy(pos_hbm, pos_v)

        def chunk(g, carry):
            start = base + g * ROWS
            pbase = (g % (SEQ_LEN // ROWS)) * ROWS
            pltpu.sync_copy(x_hbm.at[pl.ds(start, ROWS)], idx_v)
            pltpu.async_copy(tok_hbm.at[idx_v], rows_v, sem).wait()

            def addrow(i, c):
                p = pbase + i
                rows_v[i, pl.ds(0, 16)] = (
                    rows_v[i, pl.ds(0, 16)] + pos_v[p, pl.ds(0, 16)]
                )
                rows_v[i, pl.ds(16, 16)] = (
                    rows_v[i, pl.ds(16, 16)] + pos_v[p, pl.ds(16, 16)]
                )
                return c

            lax.fori_loop(0, ROWS, addrow, 0, unroll=4)
            pltpu.sync_copy(rows_v, out_hbm.at[pl.ds(start, ROWS)])
            return carry

        lax.fori_loop(0, nchunks, chunk, 0)

    return k(x_flat, tok_table, pos_table)


def kernel(x, tok_table, pos_table):
    b, l = x.shape
    x_flat = x.reshape(b * l).astype(jnp.int32)
    out = _sc_embed(x_flat, tok_table, pos_table.astype(jnp.float32))
    return out.reshape(b, l, EMBED_DIM)


# speedup vs baseline: 1.5870x; 1.5870x over previous
import functools

import jax
import jax.numpy as jnp
from jax import lax
from jax.experimental import pallas as pl
from jax.experimental.pallas import tpu as pltpu
from jax.experimental.pallas import tpu_sc as plsc

SEQ_LEN = 200
EMBED_DIM = 32
NUM_CORES = 2
NUM_SUBCORES = 16
NW = NUM_CORES * NUM_SUBCORES
ROWS = 40
NBUF = SEQ_LEN // ROWS


def _sc_embed(x_flat, tok_table, pos_table):
    n = x_flat.shape[0]
    per_w = n // NW
    nchunks = per_w // ROWS
    nouter = nchunks // NBUF
    mesh = plsc.VectorSubcoreMesh(core_axis_name="c", subcore_axis_name="s")

    scratch = (
        [pltpu.VMEM((per_w,), jnp.int32), pltpu.VMEM((SEQ_LEN, EMBED_DIM), jnp.float32)]
        + [pltpu.VMEM((ROWS, EMBED_DIM), jnp.float32) for _ in range(NBUF)]
        + [pltpu.VMEM((ROWS, EMBED_DIM), jnp.float32) for _ in range(NBUF)]
        + [pltpu.SemaphoreType.DMA for _ in range(2 * NBUF)]
    )

    @functools.partial(
        pl.kernel,
        out_type=jax.ShapeDtypeStruct((n, EMBED_DIM), jnp.float32),
        mesh=mesh,
        compiler_params=pltpu.CompilerParams(use_tc_tiling_on_sc=False),
        scratch_types=scratch,
    )
    def k(x_hbm, tok_hbm, pos_hbm, out_hbm, idx_v, pos_v, *bufs):
        gbuf = bufs[:NBUF]
        sbuf = bufs[NBUF:2 * NBUF]
        gsem = bufs[2 * NBUF:3 * NBUF]
        ssem = bufs[3 * NBUF:4 * NBUF]
        wid = lax.axis_index("s") * NUM_CORES + lax.axis_index("c")
        base = wid * per_w
        pltpu.sync_copy(pos_hbm, pos_v)
        pltpu.sync_copy(x_hbm.at[pl.ds(base, per_w)], idx_v)

        def gather_start(g, b):
            pltpu.async_copy(
                tok_hbm.at[idx_v.at[pl.ds(g * ROWS, ROWS)]], gbuf[b], gsem[b]
            )

        def gather_wait(b):
            pltpu.make_async_copy(
                tok_hbm.at[idx_v.at[pl.ds(0, ROWS)]], gbuf[b], gsem[b]
            ).wait()

        def store_start(g, b):
            pltpu.async_copy(
                sbuf[b], out_hbm.at[pl.ds(base + g * ROWS, ROWS)], ssem[b]
            )

        def store_wait(b):
            pltpu.make_async_copy(
                sbuf[b], out_hbm.at[pl.ds(base, ROWS)], ssem[b]
            ).wait()

        for b in range(NBUF):
            gather_start(b, b)

        def outer(G, carry):
            for b in range(NBUF):
                g = G * NBUF + b
                gather_wait(b)
                def addrow(i, c):
                    sbuf[b][i, pl.ds(0, 16)] = (
                        gbuf[b][i, pl.ds(0, 16)] + pos_v[b * ROWS + i, pl.ds(0, 16)]
                    )
                    sbuf[b][i, pl.ds(16, 16)] = (
                        gbuf[b][i, pl.ds(16, 16)] + pos_v[b * ROWS + i, pl.ds(16, 16)]
                    )
                    return c

                lax.fori_loop(0, ROWS, addrow, 0, unroll=8)
                @pl.when(G > 0)
                def _():
                    store_wait(b)

                store_start(g, b)
                @pl.when(G + 1 < nouter)
                def _():
                    gather_start(g + NBUF, b)

            return carry

        lax.fori_loop(0, nouter, outer, 0)
        for b in range(NBUF):
            store_wait(b)

    return k(x_flat, tok_table, pos_table)


def kernel(x, tok_table, pos_table):
    b, l = x.shape
    x_flat = x.reshape(b * l).astype(jnp.int32)
    out = _sc_embed(x_flat, tok_table, pos_table.astype(jnp.float32))
    return out.reshape(b, l, EMBED_DIM)
